# baseline (speedup 1.0000x reference)
import functools

import jax
import jax.numpy as jnp
from jax import lax
from jax.experimental import pallas as pl
from jax.experimental.pallas import tpu as pltpu
from jax.experimental.pallas import tpu_sc as plsc

EPS = 1e-12

NUM_SC_CORES = 2
NUM_SC_SUBCORES = 16
NUM_WORKERS = NUM_SC_CORES * NUM_SC_SUBCORES

CHUNK = 32
TC_TW = 512


def _sc_gather_two(word_table, pos_table, ids, pids):
    n = ids.shape[0]
    d = word_table.shape[1]
    per_w = n // NUM_WORKERS
    assert per_w % (2 * CHUNK) == 0 and per_w % 8 == 0

    mesh = plsc.VectorSubcoreMesh(core_axis_name="c", subcore_axis_name="s")
    out_sds = jax.ShapeDtypeStruct((n, d), jnp.float32)

    @functools.partial(
        pl.kernel,
        out_type=[out_sds, out_sds],
        mesh=mesh,
        scratch_types=[
            pltpu.VMEM((per_w,), jnp.int32),
            pltpu.VMEM((per_w,), jnp.int32),
            pltpu.VMEM((CHUNK, d), jnp.float32),
            pltpu.VMEM((CHUNK, d), jnp.float32),
            pltpu.VMEM((CHUNK, d), jnp.float32),
            pltpu.VMEM((CHUNK, d), jnp.float32),
            pltpu.SemaphoreType.DMA,
            pltpu.SemaphoreType.DMA,
            pltpu.SemaphoreType.DMA,
            pltpu.SemaphoreType.DMA,
            pltpu.SemaphoreType.DMA,
            pltpu.SemaphoreType.DMA,
            pltpu.SemaphoreType.DMA,
            pltpu.SemaphoreType.DMA,
        ],
    )
    def sc_kernel(wt_hbm, pt_hbm, wid_hbm, pid_hbm, ow_hbm, op_hbm,
                  widx_v, pidx_v, wbuf0, pbuf0, wbuf1, pbuf1,
                  gsw0, gsp0, gsw1, gsp1, ssw0, ssp0, ssw1, ssp1):
        wid = lax.axis_index("s") * NUM_SC_CORES + lax.axis_index("c")
        base = wid * per_w
        pltpu.sync_copy(wid_hbm.at[pl.ds(base, per_w)], widx_v)
        pltpu.sync_copy(pid_hbm.at[pl.ds(base, per_w)], pidx_v)

        bufs = ((wbuf0, pbuf0, gsw0, gsp0, ssw0, ssp0),
                (wbuf1, pbuf1, gsw1, gsp1, ssw1, ssp1))

        def issue_gather(off, b):
            wb, pb, gsw, gsp, _, _ = bufs[b]
            pltpu.async_copy(wt_hbm.at[widx_v.at[pl.ds(off, CHUNK)]], wb, gsw)
            pltpu.async_copy(pt_hbm.at[pidx_v.at[pl.ds(off, CHUNK)]], pb, gsp)

        def wait_gather(b):
            wb, pb, gsw, gsp, _, _ = bufs[b]
            pltpu.make_async_copy(wt_hbm.at[widx_v.at[pl.ds(0, CHUNK)]],
                                  wb, gsw).wait()
            pltpu.make_async_copy(pt_hbm.at[pidx_v.at[pl.ds(0, CHUNK)]],
                                  pb, gsp).wait()

        def issue_store(off, b):
            wb, pb, _, _, ssw, ssp = bufs[b]
            pltpu.async_copy(wb, ow_hbm.at[pl.ds(base + off, CHUNK)], ssw)
            pltpu.async_copy(pb, op_hbm.at[pl.ds(base + off, CHUNK)], ssp)

        def wait_store(b):
            wb, pb, _, _, ssw, ssp = bufs[b]
            pltpu.make_async_copy(wb, ow_hbm.at[pl.ds(base, CHUNK)],
                                  ssw).wait()
            pltpu.make_async_copy(pb, op_hbm.at[pl.ds(base, CHUNK)],
                                  ssp).wait()

        issue_gather(0, 0)
        issue_gather(CHUNK, 1)

        @pl.loop(0, per_w, step=2 * CHUNK)
        def _(off):
            wait_gather(0)
            issue_store(off, 0)
            wait_gather(1)
            issue_store(off + CHUNK, 1)

            @pl.when(off + 2 * CHUNK < per_w)
            def _():
                wait_store(0)
                issue_gather(off + 2 * CHUNK, 0)
                wait_store(1)
                issue_gather(off + 3 * CHUNK, 1)

        wait_store(0)
        wait_store(1)

    return sc_kernel(word_table, pos_table, ids, pids)


def _ln_body(*refs):
    if len(refs) == 6:
        _, w_ref, p_ref, g_ref, b_ref, o_ref = refs
    else:
        w_ref, p_ref, g_ref, b_ref, o_ref = refs
    x = w_ref[...] + p_ref[...]
    mean = jnp.mean(x, axis=-1, keepdims=True)
    xc = x - mean
    var = jnp.mean(xc * xc, axis=-1, keepdims=True)
    o_ref[...] = xc * lax.rsqrt(var + EPS) * g_ref[...] + b_ref[...]


def _tc_layernorm_into(acc, w_rows, p_rows, gamma, beta, row0, bs):
    n, d = w_rows.shape
    block0 = row0 // TC_TW
    row_spec = pl.BlockSpec((TC_TW, d), lambda i: (i, 0))
    vec_spec = pl.BlockSpec((1, d), lambda i: (0, 0))
    in_specs = [row_spec, row_spec, vec_spec, vec_spec]
    operands = [w_rows, p_rows, gamma.reshape(1, d), beta.reshape(1, d)]
    aliases = {}
    if acc is not None:
        in_specs.insert(0, pl.BlockSpec(memory_space=pl.ANY))
        operands.insert(0, acc)
        aliases = {0: 0}
    return pl.pallas_call(
        _ln_body,
        grid=(n // TC_TW,),
        in_specs=in_specs,
        out_specs=pl.BlockSpec((TC_TW, d), lambda i: (block0 + i, 0)),
        out_shape=jax.ShapeDtypeStruct((bs, d), jnp.float32),
        input_output_aliases=aliases,
        compiler_params=pltpu.CompilerParams(
            dimension_semantics=("parallel",)),
    )(*operands)


NCHUNK = 2


def kernel(input_ids, position_ids, word_table, pos_table, gamma, beta):
    b, s = input_ids.shape
    d = word_table.shape[1]
    bs = b * s
    ids = input_ids.reshape(-1)
    pids = position_ids.reshape(-1)

    nc = bs // NCHUNK
    gathered = []
    for k in range(NCHUNK):
        w_k, p_k = _sc_gather_two(
            word_table, pos_table,
            lax.slice(ids, (k * nc,), ((k + 1) * nc,)),
            lax.slice(pids, (k * nc,), ((k + 1) * nc,)))
        gathered.append((w_k, p_k))

    acc = None
    for k, (w_k, p_k) in enumerate(gathered):
        acc = _tc_layernorm_into(acc, w_k, p_k, gamma, beta, k * nc, bs)
    return acc.reshape(b, s, d)


# speedup vs baseline: 1.1639x; 1.1639x over previous
import functools

import jax
import jax.numpy as jnp
from jax import lax
from jax.experimental import pallas as pl
from jax.experimental.pallas import tpu as pltpu
from jax.experimental.pallas import tpu_sc as plsc

EPS = 1e-12

NUM_SC_CORES = 2
NUM_SC_SUBCORES = 16
NUM_WORKERS = NUM_SC_CORES * NUM_SC_SUBCORES
LANES = 16

CHUNK = 32
TC_TW = 512


def _sc_gather_sum(word_table, pos_table, ids, pids):
    n = ids.shape[0]
    d = word_table.shape[1]
    per_w = n // NUM_WORKERS
    assert per_w % (2 * CHUNK) == 0 and per_w % 8 == 0 and d % LANES == 0

    mesh = plsc.VectorSubcoreMesh(core_axis_name="c", subcore_axis_name="s")

    @functools.partial(
        pl.kernel,
        out_type=jax.ShapeDtypeStruct((n, d), jnp.float32),
        mesh=mesh,
        scratch_types=[
            pltpu.VMEM((per_w,), jnp.int32),
            pltpu.VMEM((per_w,), jnp.int32),
            pltpu.VMEM((CHUNK, d), jnp.float32),
            pltpu.VMEM((CHUNK, d), jnp.float32),
            pltpu.VMEM((CHUNK, d), jnp.float32),
            pltpu.VMEM((CHUNK, d), jnp.float32),
            pltpu.SemaphoreType.DMA,
            pltpu.SemaphoreType.DMA,
            pltpu.SemaphoreType.DMA,
            pltpu.SemaphoreType.DMA,
            pltpu.SemaphoreType.DMA,
            pltpu.SemaphoreType.DMA,
        ],
    )
    def sc_kernel(wt_hbm, pt_hbm, wid_hbm, pid_hbm, out_hbm,
                  widx_v, pidx_v, wbuf0, pbuf0, wbuf1, pbuf1,
                  gsw0, gsp0, gsw1, gsp1, ss0, ss1):
        wid = lax.axis_index("s") * NUM_SC_CORES + lax.axis_index("c")
        base = wid * per_w
        pltpu.sync_copy(wid_hbm.at[pl.ds(base, per_w)], widx_v)
        pltpu.sync_copy(pid_hbm.at[pl.ds(base, per_w)], pidx_v)

        bufs = ((wbuf0, pbuf0, gsw0, gsp0, ss0),
                (wbuf1, pbuf1, gsw1, gsp1, ss1))

        def issue_gather(off, b):
            wb, pb, gsw, gsp, _ = bufs[b]
            pltpu.async_copy(wt_hbm.at[widx_v.at[pl.ds(off, CHUNK)]], wb, gsw)
            pltpu.async_copy(pt_hbm.at[pidx_v.at[pl.ds(off, CHUNK)]], pb, gsp)

        def wait_gather(b):
            wb, pb, gsw, gsp, _ = bufs[b]
            pltpu.make_async_copy(wt_hbm.at[widx_v.at[pl.ds(0, CHUNK)]],
                                  wb, gsw).wait()
            pltpu.make_async_copy(pt_hbm.at[pidx_v.at[pl.ds(0, CHUNK)]],
                                  pb, gsp).wait()

        def add_rows(b):
            wb, pb, _, _, _ = bufs[b]

            @pl.loop(0, CHUNK)
            def _(r):
                for c in range(d // LANES):
                    sl = pl.ds(c * LANES, LANES)
                    plsc.addupdate(wb.at[r, sl], pb[r, sl])

        def issue_store(off, b):
            wb, _, _, _, ss = bufs[b]
            pltpu.async_copy(wb, out_hbm.at[pl.ds(base + off, CHUNK)], ss)

        def wait_store(b):
            wb, _, _, _, ss = bufs[b]
            pltpu.make_async_copy(wb, out_hbm.at[pl.ds(base, CHUNK)],
                                  ss).wait()

        issue_gather(0, 0)
        issue_gather(CHUNK, 1)

        @pl.loop(0, per_w, step=2 * CHUNK)
        def _(off):
            wait_gather(0)
            add_rows(0)
            issue_store(off, 0)
            wait_gather(1)
            add_rows(1)
            issue_store(off + CHUNK, 1)

            @pl.when(off + 2 * CHUNK < per_w)
            def _():
                wait_store(0)
                issue_gather(off + 2 * CHUNK, 0)
                wait_store(1)
                issue_gather(off + 3 * CHUNK, 1)

        wait_store(0)
        wait_store(1)

    return sc_kernel(word_table, pos_table, ids, pids)


def _ln_body(x_ref, g_ref, b_ref, o_ref):
    x = x_ref[...]
    mean = jnp.mean(x, axis=-1, keepdims=True)
    xc = x - mean
    var = jnp.mean(xc * xc, axis=-1, keepdims=True)
    o_ref[...] = xc * lax.rsqrt(var + EPS) * g_ref[...] + b_ref[...]


def _tc_layernorm(rows, gamma, beta):
    n, d = rows.shape
    row_spec = pl.BlockSpec((TC_TW, d), lambda i: (i, 0))
    vec_spec = pl.BlockSpec((1, d), lambda i: (0, 0))
    return pl.pallas_call(
        _ln_body,
        grid=(n // TC_TW,),
        in_specs=[row_spec, vec_spec, vec_spec],
        out_specs=row_spec,
        out_shape=jax.ShapeDtypeStruct((n, d), jnp.float32),
        compiler_params=pltpu.CompilerParams(
            dimension_semantics=("parallel",)),
    )(rows, gamma.reshape(1, d), beta.reshape(1, d))


def kernel(input_ids, position_ids, word_table, pos_table, gamma, beta):
    b, s = input_ids.shape
    d = word_table.shape[1]
    ids = input_ids.reshape(-1)
    pids = position_ids.reshape(-1)
    summed = _sc_gather_sum(word_table, pos_table, ids, pids)
    out = _tc_layernorm(summed, gamma, beta)
    return out.reshape(b, s, d)
